# Initial kernel scaffold; baseline (speedup 1.0000x reference)
import jax
import jax.numpy as jnp
from jax.experimental import pallas as pl


def kernel(seq, rc, expression):
    raise NotImplementedError("write your pallas kernel here")


# speedup vs baseline: 1.5753x; 1.5753x over previous
import jax
import jax.numpy as jnp
from jax.experimental import pallas as pl

_N_MUT = 1024


def _tables(length: int):
    kp = jax.random.key(42)
    kpos, kperm = jax.random.split(kp)
    pos = jax.random.randint(kpos, (_N_MUT,), 0, length)
    perm_keys = jax.random.split(kperm, _N_MUT)
    perm = jax.vmap(lambda k: jax.random.permutation(k, 4))(perm_keys).T
    base = jnp.broadcast_to(
        jnp.arange(4, dtype=jnp.int32)[:, None], (4, length))
    g = base.at[:, pos].set(perm.astype(jnp.int32))
    h = g[::-1, ::-1]
    return g, h


def _sel(tab, x0, x1, x2, x3):
    return jnp.where(tab == 0, x0,
           jnp.where(tab == 1, x1,
           jnp.where(tab == 2, x2, x3)))


def _mutate_kernel(g_ref, h_ref, seq_ref, out_ref, rc_ref):
    s = seq_ref[...]
    g = g_ref[...]
    h = h_ref[...]
    out_ref[...] = _sel(g, s[:, 0:1, :], s[:, 1:2, :], s[:, 2:3, :],
                        s[:, 3:4, :])
    row = jax.lax.broadcasted_iota(jnp.int32, (128, 128), 0)
    col = jax.lax.broadcasted_iota(jnp.int32, (128, 128), 1)
    exch = (row + col == 127).astype(jnp.float32)
    sr = [jax.lax.dot(s[:, c, :], exch,
                      preferred_element_type=jnp.float32)[:, None, :]
          for c in range(4)]
    rc_ref[...] = _sel(h, sr[0], sr[1], sr[2], sr[3])


def kernel(seq, rc, expression):
    del rc
    B, C, L = seq.shape
    assert C == 4 and L % 128 == 0
    g, h = _tables(L)
    g = g.reshape(1, C, L)
    h = h.reshape(1, C, L)
    nj = L // 128
    out_seq, out_rc = pl.pallas_call(
        _mutate_kernel,
        grid=(nj,),
        in_specs=[
            pl.BlockSpec((1, C, 128), lambda j: (0, 0, j)),
            pl.BlockSpec((1, C, 128), lambda j: (0, 0, nj - 1 - j)),
            pl.BlockSpec((B, C, 128), lambda j: (0, 0, j)),
        ],
        out_specs=[
            pl.BlockSpec((B, C, 128), lambda j: (0, 0, j)),
            pl.BlockSpec((B, C, 128), lambda j: (0, 0, nj - 1 - j)),
        ],
        out_shape=[
            jax.ShapeDtypeStruct(seq.shape, seq.dtype),
            jax.ShapeDtypeStruct(seq.shape, seq.dtype),
        ],
    )(g, h, seq)
    return (out_seq, out_rc, expression)


# speedup vs baseline: 7.7929x; 4.9468x over previous
import functools

import jax
import jax.numpy as jnp
import numpy as np
from jax.experimental import pallas as pl

_N_MUT = 1024
_LB = 2048


@functools.lru_cache(maxsize=None)
def _tables(length: int):
    with jax.ensure_compile_time_eval():
        kp = jax.random.key(42)
        kpos, kperm = jax.random.split(kp)
        pos = np.asarray(jax.random.randint(kpos, (_N_MUT,), 0, length))
        perm_keys = jax.random.split(kperm, _N_MUT)
        perm = np.asarray(
            jax.vmap(lambda k: jax.random.permutation(k, 4))(perm_keys).T)
    g = np.tile(np.arange(4, dtype=np.int32)[:, None], (1, length))
    g[:, pos] = perm.astype(np.int32)
    return g


def _mutate_kernel(g_ref, seq_ref, out_ref, rc_ref):
    s = seq_ref[...]
    g = g_ref[...]
    out = jnp.where(g == 0, s[:, 0:1, :],
          jnp.where(g == 1, s[:, 1:2, :],
          jnp.where(g == 2, s[:, 2:3, :], s[:, 3:4, :])))
    out_ref[...] = out
    row = jax.lax.broadcasted_iota(jnp.int32, (128, 128), 0)
    col = jax.lax.broadcasted_iota(jnp.int32, (128, 128), 1)
    exch = (row + col == 127).astype(jnp.float32)
    nk = _LB // 128
    planes = []
    for c in range(4):
        v = out[:, 3 - c, :]
        parts = [jax.lax.dot(v[:, k * 128:(k + 1) * 128], exch,
                             preferred_element_type=jnp.float32)
                 for k in range(nk)]
        planes.append(jnp.concatenate(parts[::-1], axis=1)[:, None, :])
    rc_ref[...] = jnp.concatenate(planes, axis=1)


def kernel(seq, rc, expression):
    del rc
    B, C, L = seq.shape
    assert C == 4 and L % _LB == 0
    g = jnp.asarray(_tables(L)).reshape(1, C, L)
    nj = L // _LB
    out_seq, out_rc = pl.pallas_call(
        _mutate_kernel,
        grid=(nj,),
        in_specs=[
            pl.BlockSpec((1, C, _LB), lambda j: (0, 0, j)),
            pl.BlockSpec((B, C, _LB), lambda j: (0, 0, j)),
        ],
        out_specs=[
            pl.BlockSpec((B, C, _LB), lambda j: (0, 0, j)),
            pl.BlockSpec((B, C, _LB), lambda j: (0, 0, nj - 1 - j)),
        ],
        out_shape=[
            jax.ShapeDtypeStruct(seq.shape, seq.dtype),
            jax.ShapeDtypeStruct(seq.shape, seq.dtype),
        ],
    )(g, seq)
    return (out_seq, out_rc, expression)


# speedup vs baseline: 13.5762x; 1.7421x over previous
import functools

import jax
import jax.numpy as jnp
import numpy as np
from jax.experimental import pallas as pl

_N_MUT = 1024
_LB = 2048


@functools.lru_cache(maxsize=None)
def _tables(length: int):
    with jax.ensure_compile_time_eval():
        kp = jax.random.key(42)
        kpos, kperm = jax.random.split(kp)
        pos = np.asarray(jax.random.randint(kpos, (_N_MUT,), 0, length))
        perm_keys = jax.random.split(kperm, _N_MUT)
        perm = np.asarray(
            jax.vmap(lambda k: jax.random.permutation(k, 4))(perm_keys).T)
    g = np.tile(np.arange(4, dtype=np.int32)[:, None], (1, length))
    g[:, pos] = perm.astype(np.int32)
    return g


def _mutate_kernel(g_ref, h_ref, seq_ref, out_ref, rc_ref):
    s = seq_ref[...]
    gb = jnp.broadcast_to(g_ref[...], s.shape)
    out_ref[...] = jnp.take_along_axis(s, gb, axis=1)
    row = jax.lax.broadcasted_iota(jnp.int32, (128, 128), 0)
    col = jax.lax.broadcasted_iota(jnp.int32, (128, 128), 1)
    exch = (row + col == 127).astype(jnp.float32)
    nk = _LB // 128
    chunks = [jax.lax.dot_general(
        s[:, :, k * 128:(k + 1) * 128], exch,
        (((2,), (0,)), ((), ())), preferred_element_type=jnp.float32)
        for k in range(nk)]
    sr = jnp.concatenate(chunks[::-1], axis=2)
    hb = jnp.broadcast_to(h_ref[...], s.shape)
    rc_ref[...] = jnp.take_along_axis(sr, hb, axis=1)


def kernel(seq, rc, expression):
    del rc
    B, C, L = seq.shape
    assert C == 4 and L % _LB == 0
    g_np = _tables(L)
    h_np = g_np[::-1, ::-1].copy()
    g = jnp.asarray(g_np).reshape(1, C, L)
    h = jnp.asarray(h_np).reshape(1, C, L)
    nj = L // _LB
    out_seq, out_rc = pl.pallas_call(
        _mutate_kernel,
        grid=(nj,),
        in_specs=[
            pl.BlockSpec((1, C, _LB), lambda j: (0, 0, j)),
            pl.BlockSpec((1, C, _LB), lambda j: (0, 0, nj - 1 - j)),
            pl.BlockSpec((B, C, _LB), lambda j: (0, 0, j)),
        ],
        out_specs=[
            pl.BlockSpec((B, C, _LB), lambda j: (0, 0, j)),
            pl.BlockSpec((B, C, _LB), lambda j: (0, 0, nj - 1 - j)),
        ],
        out_shape=[
            jax.ShapeDtypeStruct(seq.shape, seq.dtype),
            jax.ShapeDtypeStruct(seq.shape, seq.dtype),
        ],
    )(g, h, seq)
    return (out_seq, out_rc, expression)


# speedup vs baseline: 14.3959x; 1.0604x over previous
import functools

import jax
import jax.numpy as jnp
import numpy as np
from jax.experimental import pallas as pl

_N_MUT = 1024
_LB = 4096


@functools.lru_cache(maxsize=None)
def _tables(length: int):
    with jax.ensure_compile_time_eval():
        kp = jax.random.key(42)
        kpos, kperm = jax.random.split(kp)
        pos = np.asarray(jax.random.randint(kpos, (_N_MUT,), 0, length))
        perm_keys = jax.random.split(kperm, _N_MUT)
        perm = np.asarray(
            jax.vmap(lambda k: jax.random.permutation(k, 4))(perm_keys).T)
    g = np.tile(np.arange(4, dtype=np.int32)[:, None], (1, length))
    g[:, pos] = perm.astype(np.int32)
    return g


def _mutate_kernel(g_ref, h_ref, seq_ref, out_ref, rc_ref):
    s = seq_ref[...]
    gb = jnp.broadcast_to(g_ref[...], s.shape)
    out_ref[...] = jnp.take_along_axis(s, gb, axis=1)
    row = jax.lax.broadcasted_iota(jnp.int32, (128, 128), 0)
    col = jax.lax.broadcasted_iota(jnp.int32, (128, 128), 1)
    exch = (row + col == 127).astype(jnp.float32)
    nk = _LB // 128
    chunks = [jax.lax.dot_general(
        s[:, :, k * 128:(k + 1) * 128], exch,
        (((2,), (0,)), ((), ())), preferred_element_type=jnp.float32)
        for k in range(nk)]
    sr = jnp.concatenate(chunks[::-1], axis=2)
    hb = jnp.broadcast_to(h_ref[...], s.shape)
    rc_ref[...] = jnp.take_along_axis(sr, hb, axis=1)


def kernel(seq, rc, expression):
    del rc
    B, C, L = seq.shape
    assert C == 4 and L % _LB == 0
    g_np = _tables(L)
    h_np = g_np[::-1, ::-1].copy()
    g = jnp.asarray(g_np).reshape(1, C, L)
    h = jnp.asarray(h_np).reshape(1, C, L)
    nj = L // _LB
    out_seq, out_rc = pl.pallas_call(
        _mutate_kernel,
        grid=(nj,),
        in_specs=[
            pl.BlockSpec((1, C, _LB), lambda j: (0, 0, j)),
            pl.BlockSpec((1, C, _LB), lambda j: (0, 0, nj - 1 - j)),
            pl.BlockSpec((B, C, _LB), lambda j: (0, 0, j)),
        ],
        out_specs=[
            pl.BlockSpec((B, C, _LB), lambda j: (0, 0, j)),
            pl.BlockSpec((B, C, _LB), lambda j: (0, 0, nj - 1 - j)),
        ],
        out_shape=[
            jax.ShapeDtypeStruct(seq.shape, seq.dtype),
            jax.ShapeDtypeStruct(seq.shape, seq.dtype),
        ],
    )(g, h, seq)
    return (out_seq, out_rc, expression)
